# Initial kernel scaffold; baseline (speedup 1.0000x reference)
import jax
import jax.numpy as jnp
from jax.experimental import pallas as pl


def kernel(x, edge_index_0, edge_index_1, edge_index_2, edge_index_3, edge_index_4, edge_index_5, edge_attr_0, edge_attr_1, edge_attr_2, edge_attr_3, edge_attr_4, edge_attr_5, W1, att1, W2, att2, gamma1, beta1, gamma2, beta2, query, W_out, b_out):
    raise NotImplementedError("write your pallas kernel here")


# speedup vs baseline: 18.7701x; 18.7701x over previous
import functools

import jax
import jax.numpy as jnp
from jax import lax
from jax.experimental import pallas as pl
from jax.experimental.pallas import tpu as pltpu
from jax.experimental.pallas import tpu_sc as plsc

_F32 = jnp.float32


def _bf16_dot(a, b):
    return jax.lax.dot_general(a.astype(jnp.bfloat16), b.astype(jnp.bfloat16),
                               (((1,), (0,)), ((), ())),
                               preferred_element_type=_F32)


def _dense_body(x_ref, w_ref, att_ref, xh_ref, a_ref):
    xw = _bf16_dot(x_ref[...], w_ref[0])
    xh_ref[0] = xw
    d = x_ref.shape[1]
    ahs = [jnp.sum(xw[:, h * d:(h + 1) * d] * att_ref[0, h, :][None, :],
                   axis=1, keepdims=True)
           for h in range(2)]
    a_ref[0] = jnp.concatenate(ahs, axis=1)


def _dense(xin, w, att):
    npad, d = xin.shape
    t = w.shape[0]
    bn = 256
    return pl.pallas_call(
        _dense_body,
        grid=(t, npad // bn),
        in_specs=[
            pl.BlockSpec((bn, d), lambda ti, i: (i, 0)),
            pl.BlockSpec((1, d, 2 * d), lambda ti, i: (ti, 0, 0)),
            pl.BlockSpec((1, 2, d), lambda ti, i: (ti, 0, 0)),
        ],
        out_specs=[
            pl.BlockSpec((1, bn, 2 * d), lambda ti, i: (ti, i, 0)),
            pl.BlockSpec((1, bn, 2), lambda ti, i: (ti, i, 0)),
        ],
        out_shape=[
            jax.ShapeDtypeStruct((t, npad, 2 * d), _F32),
            jax.ShapeDtypeStruct((t, npad, 2), _F32),
        ],
    )(xin, w, att)


def _norm_body(raw_ref, g_ref, b_ref, out_ref):
    hb = jnp.concatenate([raw_ref[0], raw_ref[1], raw_ref[2], raw_ref[3]], axis=-1)
    r = jnp.maximum(hb, 0.0)
    mu = jnp.mean(r, axis=-1, keepdims=True)
    var = jnp.mean((r - mu) ** 2, axis=-1, keepdims=True)
    out_ref[...] = (r - mu) / jnp.sqrt(var + 1e-5) * g_ref[...] + b_ref[...]


def _norm(raw, g, b):
    _, npad, dq = raw.shape
    d = 4 * dq
    bn = 256
    return pl.pallas_call(
        _norm_body,
        grid=(npad // bn,),
        in_specs=[
            pl.BlockSpec((4, bn, dq), lambda i: (0, i, 0)),
            pl.BlockSpec((d,), lambda i: (0,)),
            pl.BlockSpec((d,), lambda i: (0,)),
        ],
        out_specs=pl.BlockSpec((bn, d), lambda i: (i, 0)),
        out_shape=jax.ShapeDtypeStruct((npad, d), _F32),
    )(raw, g, b)


def _pool_body(n_valid, bn, nb, h_ref, q_ref, wo_ref, bo_ref, out_ref,
               r_acc, mz_acc):
    i = pl.program_id(0)
    d = h_ref.shape[1]

    @pl.when(i == 0)
    def _():
        r_acc[...] = jnp.zeros_like(r_acc)
        mz_acc[0, 0] = -1e30
        mz_acc[0, 1] = 0.0

    s = _bf16_dot(h_ref[...], q_ref[...].reshape(d, 1)).reshape(bn // 128, 128)
    rows = lax.broadcasted_iota(jnp.int32, s.shape, 0)
    cols = lax.broadcasted_iota(jnp.int32, s.shape, 1)
    valid = (i * bn + rows * 128 + cols) < n_valid
    sm = jnp.where(valid, s, -1e30)
    m_prev = mz_acc[0, 0]
    z_prev = mz_acc[0, 1]
    m_new = jnp.maximum(m_prev, jnp.max(sm))
    scale = jnp.exp(m_prev - m_new)
    e = jnp.where(valid, jnp.exp(sm - m_new), 0.0)
    z_new = jnp.where(i == 0, 0.0, z_prev * scale) + jnp.sum(e)
    ev = e.reshape(1, bn)
    r_new = r_acc[...] * scale + jax.lax.dot_general(
        ev, h_ref[...], (((1,), (0,)), ((), ())),
        precision=lax.Precision.HIGHEST)
    r_acc[...] = r_new
    mz_acc[0, 0] = m_new
    mz_acc[0, 1] = z_new

    @pl.when(i == nb - 1)
    def _():
        rep = r_acc[...] / z_new
        out_ref[...] = _bf16_dot(rep, wo_ref[...]) + bo_ref[...].reshape(1, d)


def _pool(h2, q, wo, bo, n_valid):
    npad, d = h2.shape
    bn = 1792 if npad % 1792 == 0 else 256
    nb = npad // bn
    out = pl.pallas_call(
        functools.partial(_pool_body, n_valid, bn, nb),
        grid=(nb,),
        in_specs=[
            pl.BlockSpec((bn, d), lambda i: (i, 0)),
            pl.BlockSpec((d,), lambda i: (0,)),
            pl.BlockSpec((d, d), lambda i: (0, 0)),
            pl.BlockSpec((d,), lambda i: (0,)),
        ],
        out_specs=pl.BlockSpec((1, d), lambda i: (0, 0)),
        out_shape=jax.ShapeDtypeStruct((1, d), _F32),
        scratch_shapes=[pltpu.VMEM((1, d), _F32), pltpu.SMEM((1, 2), _F32)],
    )(h2, q, wo, bo)
    return out[0]


def _alpha_call(a, src, dst, ea):
    two, t, npad = a.shape
    p = src.shape[1]
    nps = npad // 16
    pj = p // 16
    mesh = plsc.VectorSubcoreMesh(core_axis_name="c", subcore_axis_name="s", num_cores=2, num_subcores=16)

    @functools.partial(
        pl.kernel,
        out_type=jax.ShapeDtypeStruct((t, 2, p, 128), _F32),
        mesh=mesh,
        compiler_params=pltpu.CompilerParams(needs_layout_passes=False, use_tc_tiling_on_sc=False),
        scratch_types=[
            pltpu.VMEM_SHARED((npad,), _F32),
            pltpu.VMEM_SHARED((p, 128), _F32),
            pltpu.VMEM((npad,), _F32),
            pltpu.VMEM((128,), jnp.int32),
            pltpu.VMEM((1, 128), jnp.int32),
            pltpu.VMEM((128,), _F32),
            pltpu.VMEM((1, 128), _F32),
            pltpu.VMEM((1, 128), _F32),
            pltpu.VMEM((nps,), _F32),
        ],
    )
    def k(a_hbm, src_hbm, dst_hbm, ea_hbm, alpha_hbm,
          den_s, exs_s, tbl, srcb, dstb, eab, exb, alb, zb):
        c = lax.axis_index("c")
        s = lax.axis_index("s")

        def zfill(i, _):
            zb[pl.ds(i * 16, 16)] = jnp.zeros((16,), _F32)
            return 0
        lax.fori_loop(0, nps // 16, zfill, 0)

        for ti in range(t):
            pltpu.sync_copy(zb, den_s.at[pl.ds(s * nps, nps)])
            pltpu.sync_copy(a_hbm.at[c, ti], tbl)
            plsc.subcore_barrier()

            def panel_a(j, _):
                jg = s * pj + j
                pltpu.sync_copy(src_hbm.at[ti, jg], srcb)
                pltpu.sync_copy(dst_hbm.at[ti, jg], dstb.at[0])
                pltpu.sync_copy(ea_hbm.at[ti, jg], eab)
                for kk in range(8):
                    sl = pl.ds(kk * 16, 16)
                    sv = srcb[sl]
                    dv = dstb[0, sl]
                    aj = plsc.load_gather(tbl, [sv])
                    ai = plsc.load_gather(tbl, [dv])
                    ss = ai + aj
                    lk = jnp.where(ss > 0, ss, 0.2 * ss)
                    exb[0, sl] = jnp.exp(lk * eab[sl])
                pltpu.sync_copy(exb.at[0], exs_s.at[jg])
                pltpu.sync_copy(exb.at[0], den_s.at[dstb.at[0]], add=True)
                return 0
            lax.fori_loop(0, pj, panel_a, 0)
            plsc.subcore_barrier()

            pltpu.sync_copy(den_s, tbl)

            def panel_b(j, _):
                jg = s * pj + j
                pltpu.sync_copy(dst_hbm.at[ti, jg], dstb.at[0])
                pltpu.sync_copy(exs_s.at[jg], exb.at[0])
                for kk in range(8):
                    sl = pl.ds(kk * 16, 16)
                    dv = dstb[0, sl]
                    dn = plsc.load_gather(tbl, [dv])
                    alb[0, sl] = 0.5 * exb[0, sl] / (dn + 1e-16)
                pltpu.sync_copy(alb.at[0], alpha_hbm.at[ti, c, jg])
                return 0
            lax.fori_loop(0, pj, panel_b, 0)
            plsc.subcore_barrier()

    return k(a, src, dst, ea)


def _message_call(xh_flat, src, dst, alpha, npad):
    t, p, _ = src.shape
    nps = npad // 16
    zr = nps // 16
    pj = p // 16
    np8 = npad * 8
    mesh = plsc.VectorSubcoreMesh(core_axis_name="c", subcore_axis_name="s", num_cores=2, num_subcores=16)

    @functools.partial(
        pl.kernel,
        out_type=jax.ShapeDtypeStruct((4, npad, 32), _F32),
        mesh=mesh,
        compiler_params=pltpu.CompilerParams(needs_layout_passes=False, use_tc_tiling_on_sc=False),
        scratch_types=[
            pltpu.VMEM_SHARED((npad, 32), _F32),
            pltpu.VMEM((128, 32), _F32),
            pltpu.VMEM((128, 32), _F32),
            pltpu.VMEM((128, 32), _F32),
            pltpu.VMEM((128,), jnp.int32),
            pltpu.VMEM((1, 128), jnp.int32),
            pltpu.VMEM((128,), jnp.int32),
            pltpu.VMEM((128,), jnp.int32),
            pltpu.VMEM((2, 128), _F32),
            pltpu.VMEM((zr, 32), _F32),
        ],
    )
    def k(xh_hbm, src_hbm, dst_hbm, al_hbm, outq_hbm,
          outs, r0, r1, msg, srcb, dstb, i0b, i1b, alb, zb):
        c = lax.axis_index("c")
        s = lax.axis_index("s")

        def zfill(i, _):
            zb[i, pl.ds(0, 16)] = jnp.zeros((16,), _F32)
            zb[i, pl.ds(16, 16)] = jnp.zeros((16,), _F32)
            return 0
        lax.fori_loop(0, zr, zfill, 0)

        for qq in range(2):
            q = 2 * c + qq
            for z in range(16):
                pltpu.sync_copy(zb, outs.at[pl.ds(s * nps + z * zr, zr)])
            plsc.subcore_barrier()
            for ti in range(t):
                base_t = ti * np8

                def panel(j, _):
                    jg = s * pj + j
                    pltpu.sync_copy(src_hbm.at[ti, jg], srcb)
                    pltpu.sync_copy(dst_hbm.at[ti, jg], dstb.at[0])
                    pltpu.sync_copy(al_hbm.at[ti, 0, jg], alb.at[0])
                    pltpu.sync_copy(al_hbm.at[ti, 1, jg], alb.at[1])
                    for kk in range(8):
                        sl = pl.ds(kk * 16, 16)
                        sv = srcb[sl]
                        i0 = base_t + sv * 8 + q
                        i0b[sl] = i0
                        i1b[sl] = i0 + 4
                    pltpu.sync_copy(xh_hbm.at[i0b], r0)
                    pltpu.sync_copy(xh_hbm.at[i1b], r1)

                    def edge(i, _):
                        i16 = (i >> 4) << 4
                        lane = jnp.full((16,), i & 15, jnp.int32)
                        a0 = jnp.take(alb[0, pl.ds(i16, 16)], lane)
                        a1 = jnp.take(alb[1, pl.ds(i16, 16)], lane)
                        for p2 in range(2):
                            sl = pl.ds(p2 * 16, 16)
                            msg[i, sl] = a0 * r0[i, sl] + a1 * r1[i, sl]
                        return 0
                    lax.fori_loop(0, 128, edge, 0)
                    pltpu.sync_copy(msg, outs.at[dstb.at[0]], add=True)
                    return 0
                lax.fori_loop(0, pj, panel, 0)
            plsc.subcore_barrier()
            for z in range(16):
                rs = s * nps + z * zr
                pltpu.sync_copy(outs.at[pl.ds(rs, zr)], outq_hbm.at[q, pl.ds(rs, zr)])
            plsc.subcore_barrier()

    return k(xh_flat, src, dst, alpha)


def _gat_layer(xin, w, att, src, dst, ea, npad):
    xh, a = _dense(xin, w, att)
    alpha = _alpha_call(jnp.transpose(a, (2, 0, 1)), src, dst, ea)
    t = w.shape[0]
    xh_flat = xh.reshape(t * npad * 8, 32)
    return _message_call(xh_flat, src, dst, alpha, npad)


def kernel(x, edge_index_0, edge_index_1, edge_index_2, edge_index_3,
           edge_index_4, edge_index_5, edge_attr_0, edge_attr_1, edge_attr_2,
           edge_attr_3, edge_attr_4, edge_attr_5, W1, att1, W2, att2,
           gamma1, beta1, gamma2, beta2, query, W_out, b_out):
    n, d = x.shape
    eis = [edge_index_0, edge_index_1, edge_index_2, edge_index_3,
           edge_index_4, edge_index_5]
    eas = [edge_attr_0, edge_attr_1, edge_attr_2, edge_attr_3,
           edge_attr_4, edge_attr_5]
    e = eis[0].shape[1]
    npad = ((n + 255) // 256) * 256
    p = e // 128

    src = jnp.stack([ei[0] for ei in eis]).reshape(6, p, 128)
    dst = jnp.stack([ei[1] for ei in eis]).reshape(6, p, 128)
    ea = jnp.stack([a[:, 0] for a in eas]).reshape(6, p, 128)
    xp = jnp.pad(x, ((0, npad - n), (0, 0)))

    raw1 = _gat_layer(xp, W1, att1, src, dst, ea, npad)
    h = _norm(raw1, gamma1, beta1)
    raw2 = _gat_layer(h, W2, att2, src, dst, ea, npad)
    h2 = _norm(raw2, gamma2, beta2)
    return _pool(h2, query, W_out, b_out, n)


# speedup vs baseline: 26.8190x; 1.4288x over previous
import functools

import jax
import jax.numpy as jnp
from jax import lax
from jax.experimental import pallas as pl
from jax.experimental.pallas import tpu as pltpu
from jax.experimental.pallas import tpu_sc as plsc

_F32 = jnp.float32


def _bf16_dot(a, b):
    return jax.lax.dot_general(a.astype(jnp.bfloat16), b.astype(jnp.bfloat16),
                               (((1,), (0,)), ((), ())),
                               preferred_element_type=_F32)


def _dense_body(x_ref, w_ref, att_ref, xh_ref, a_ref):
    xw = _bf16_dot(x_ref[...], w_ref[0])
    xh_ref[0] = xw
    d = x_ref.shape[1]
    ahs = [jnp.sum(xw[:, h * d:(h + 1) * d] * att_ref[0, h, :][None, :],
                   axis=1, keepdims=True)
           for h in range(2)]
    a_ref[0] = jnp.concatenate(ahs, axis=1)


def _dense(xin, w, att):
    npad, d = xin.shape
    t = w.shape[0]
    bn = 256
    return pl.pallas_call(
        _dense_body,
        grid=(t, npad // bn),
        in_specs=[
            pl.BlockSpec((bn, d), lambda ti, i: (i, 0)),
            pl.BlockSpec((1, d, 2 * d), lambda ti, i: (ti, 0, 0)),
            pl.BlockSpec((1, 2, d), lambda ti, i: (ti, 0, 0)),
        ],
        out_specs=[
            pl.BlockSpec((1, bn, 2 * d), lambda ti, i: (ti, i, 0)),
            pl.BlockSpec((1, bn, 2), lambda ti, i: (ti, i, 0)),
        ],
        out_shape=[
            jax.ShapeDtypeStruct((t, npad, 2 * d), _F32),
            jax.ShapeDtypeStruct((t, npad, 2), _F32),
        ],
    )(xin, w, att)


def _norm_body(raw_ref, g_ref, b_ref, out_ref):
    hb = jnp.concatenate([raw_ref[0], raw_ref[1], raw_ref[2], raw_ref[3]], axis=-1)
    r = jnp.maximum(hb, 0.0)
    mu = jnp.mean(r, axis=-1, keepdims=True)
    var = jnp.mean((r - mu) ** 2, axis=-1, keepdims=True)
    out_ref[...] = (r - mu) / jnp.sqrt(var + 1e-5) * g_ref[...] + b_ref[...]


def _norm(raw, g, b):
    _, npad, dq = raw.shape
    d = 4 * dq
    bn = 256
    return pl.pallas_call(
        _norm_body,
        grid=(npad // bn,),
        in_specs=[
            pl.BlockSpec((4, bn, dq), lambda i: (0, i, 0)),
            pl.BlockSpec((d,), lambda i: (0,)),
            pl.BlockSpec((d,), lambda i: (0,)),
        ],
        out_specs=pl.BlockSpec((bn, d), lambda i: (i, 0)),
        out_shape=jax.ShapeDtypeStruct((npad, d), _F32),
    )(raw, g, b)


def _pool_body(n_valid, bn, nb, h_ref, q_ref, wo_ref, bo_ref, out_ref,
               r_acc, mz_acc):
    i = pl.program_id(0)
    d = h_ref.shape[1]

    @pl.when(i == 0)
    def _():
        r_acc[...] = jnp.zeros_like(r_acc)
        mz_acc[0, 0] = -1e30
        mz_acc[0, 1] = 0.0

    s = _bf16_dot(h_ref[...], q_ref[...].reshape(d, 1)).reshape(bn // 128, 128)
    rows = lax.broadcasted_iota(jnp.int32, s.shape, 0)
    cols = lax.broadcasted_iota(jnp.int32, s.shape, 1)
    valid = (i * bn + rows * 128 + cols) < n_valid
    sm = jnp.where(valid, s, -1e30)
    m_prev = mz_acc[0, 0]
    z_prev = mz_acc[0, 1]
    m_new = jnp.maximum(m_prev, jnp.max(sm))
    scale = jnp.exp(m_prev - m_new)
    e = jnp.where(valid, jnp.exp(sm - m_new), 0.0)
    z_new = jnp.where(i == 0, 0.0, z_prev * scale) + jnp.sum(e)
    ev = e.reshape(1, bn)
    r_new = r_acc[...] * scale + jax.lax.dot_general(
        ev, h_ref[...], (((1,), (0,)), ((), ())),
        precision=lax.Precision.HIGHEST)
    r_acc[...] = r_new
    mz_acc[0, 0] = m_new
    mz_acc[0, 1] = z_new

    @pl.when(i == nb - 1)
    def _():
        rep = r_acc[...] / z_new
        out_ref[...] = _bf16_dot(rep, wo_ref[...]) + bo_ref[...].reshape(1, d)


def _pool(h2, q, wo, bo, n_valid):
    npad, d = h2.shape
    bn = 1792 if npad % 1792 == 0 else 256
    nb = npad // bn
    out = pl.pallas_call(
        functools.partial(_pool_body, n_valid, bn, nb),
        grid=(nb,),
        in_specs=[
            pl.BlockSpec((bn, d), lambda i: (i, 0)),
            pl.BlockSpec((d,), lambda i: (0,)),
            pl.BlockSpec((d, d), lambda i: (0, 0)),
            pl.BlockSpec((d,), lambda i: (0,)),
        ],
        out_specs=pl.BlockSpec((1, d), lambda i: (0, 0)),
        out_shape=jax.ShapeDtypeStruct((1, d), _F32),
        scratch_shapes=[pltpu.VMEM((1, d), _F32), pltpu.SMEM((1, 2), _F32)],
    )(h2, q, wo, bo)
    return out[0]


def _alpha_call(a, src, dst, ea):
    two, t, npad = a.shape
    p = src.shape[1]
    nps = npad // 16
    pj = p // 16
    mesh = plsc.VectorSubcoreMesh(core_axis_name="c", subcore_axis_name="s", num_cores=2, num_subcores=16)

    @functools.partial(
        pl.kernel,
        out_type=jax.ShapeDtypeStruct((t, 2, p, 128), _F32),
        mesh=mesh,
        compiler_params=pltpu.CompilerParams(needs_layout_passes=False, use_tc_tiling_on_sc=False),
        scratch_types=[
            pltpu.VMEM_SHARED((npad,), _F32),
            pltpu.VMEM_SHARED((p, 128), _F32),
            pltpu.VMEM((npad,), _F32),
            pltpu.VMEM((128,), jnp.int32),
            pltpu.VMEM((1, 128), jnp.int32),
            pltpu.VMEM((128,), _F32),
            pltpu.VMEM((1, 128), _F32),
            pltpu.VMEM((1, 128), _F32),
            pltpu.VMEM((nps,), _F32),
            pltpu.SemaphoreType.DMA,
            pltpu.SemaphoreType.DMA,
            pltpu.SemaphoreType.DMA,
        ],
    )
    def k(a_hbm, src_hbm, dst_hbm, ea_hbm, alpha_hbm,
          den_s, exs_s, tbl, srcb, dstb, eab, exb, alb, zb,
          sm0, sm1, sm2):
        c = lax.axis_index("c")
        s = lax.axis_index("s")

        def zfill(i, _):
            zb[pl.ds(i * 16, 16)] = jnp.zeros((16,), _F32)
            return 0
        lax.fori_loop(0, nps // 16, zfill, 0)

        for ti in range(t):
            pltpu.sync_copy(zb, den_s.at[pl.ds(s * nps, nps)])
            pltpu.sync_copy(a_hbm.at[c, ti], tbl)
            plsc.subcore_barrier()

            def panel_a(j, _):
                jg = s * pj + j
                c0 = pltpu.async_copy(src_hbm.at[ti, jg], srcb, sm0)
                c1 = pltpu.async_copy(dst_hbm.at[ti, jg], dstb.at[0], sm1)
                c2 = pltpu.async_copy(ea_hbm.at[ti, jg], eab, sm2)
                c0.wait()
                c1.wait()
                c2.wait()
                for kk in range(8):
                    sl = pl.ds(kk * 16, 16)
                    sv = srcb[sl]
                    dv = dstb[0, sl]
                    aj = plsc.load_gather(tbl, [sv])
                    ai = plsc.load_gather(tbl, [dv])
                    ss = ai + aj
                    lk = jnp.where(ss > 0, ss, 0.2 * ss)
                    exb[0, sl] = jnp.exp(lk * eab[sl])
                pltpu.sync_copy(exb.at[0], exs_s.at[jg])
                pltpu.sync_copy(exb.at[0], den_s.at[dstb.at[0]], add=True)
                return 0
            lax.fori_loop(0, pj, panel_a, 0)
            plsc.subcore_barrier()

            pltpu.sync_copy(den_s, tbl)

            def panel_b(j, _):
                jg = s * pj + j
                c0 = pltpu.async_copy(dst_hbm.at[ti, jg], dstb.at[0], sm0)
                c1 = pltpu.async_copy(exs_s.at[jg], exb.at[0], sm1)
                c0.wait()
                c1.wait()
                for kk in range(8):
                    sl = pl.ds(kk * 16, 16)
                    dv = dstb[0, sl]
                    dn = plsc.load_gather(tbl, [dv])
                    alb[0, sl] = 0.5 * exb[0, sl] / (dn + 1e-16)
                pltpu.sync_copy(alb.at[0], alpha_hbm.at[ti, c, jg])
                return 0
            lax.fori_loop(0, pj, panel_b, 0)
            plsc.subcore_barrier()

    return k(a, src, dst, ea)


def _message_call(xh_flat, src, dst, alpha, npad):
    t, p, _ = src.shape
    nps = npad // 16
    zr = nps // 16
    pj = p // 16
    np8 = npad * 8
    mesh = plsc.VectorSubcoreMesh(core_axis_name="c", subcore_axis_name="s", num_cores=2, num_subcores=16)

    @functools.partial(
        pl.kernel,
        out_type=jax.ShapeDtypeStruct((4, npad, 32), _F32),
        mesh=mesh,
        compiler_params=pltpu.CompilerParams(needs_layout_passes=False, use_tc_tiling_on_sc=False),
        scratch_types=[
            pltpu.VMEM_SHARED((npad, 32), _F32),
            pltpu.VMEM((128, 32), _F32),
            pltpu.VMEM((128, 32), _F32),
            pltpu.VMEM((128, 32), _F32),
            pltpu.VMEM((128,), jnp.int32),
            pltpu.VMEM((1, 128), jnp.int32),
            pltpu.VMEM((128,), jnp.int32),
            pltpu.VMEM((128,), jnp.int32),
            pltpu.VMEM((2, 128), _F32),
            pltpu.VMEM((zr, 32), _F32),
            pltpu.SemaphoreType.DMA,
            pltpu.SemaphoreType.DMA,
            pltpu.SemaphoreType.DMA,
            pltpu.SemaphoreType.DMA,
            pltpu.SemaphoreType.DMA,
            pltpu.SemaphoreType.DMA,
        ],
    )
    def k(xh_hbm, src_hbm, dst_hbm, al_hbm, outq_hbm,
          outs, r0, r1, msg, srcb, dstb, i0b, i1b, alb, zb,
          sm0, sm1, sm2, sm3, sg0, sg1):
        c = lax.axis_index("c")
        s = lax.axis_index("s")

        def zfill(i, _):
            zb[i, pl.ds(0, 16)] = jnp.zeros((16,), _F32)
            zb[i, pl.ds(16, 16)] = jnp.zeros((16,), _F32)
            return 0
        lax.fori_loop(0, zr, zfill, 0)

        for qq in range(2):
            q = 2 * c + qq
            for z in range(16):
                pltpu.sync_copy(zb, outs.at[pl.ds(s * nps + z * zr, zr)])
            plsc.subcore_barrier()
            for ti in range(t):
                base_t = ti * np8

                def panel(j, _):
                    jg = s * pj + j
                    cs = pltpu.async_copy(src_hbm.at[ti, jg], srcb, sm0)
                    cd = pltpu.async_copy(dst_hbm.at[ti, jg], dstb.at[0], sm1)
                    c0 = pltpu.async_copy(al_hbm.at[ti, 0, jg], alb.at[0], sm2)
                    c1 = pltpu.async_copy(al_hbm.at[ti, 1, jg], alb.at[1], sm3)
                    cs.wait()
                    for kk in range(8):
                        sl = pl.ds(kk * 16, 16)
                        sv = srcb[sl]
                        i0 = base_t + sv * 8 + q
                        i0b[sl] = i0
                        i1b[sl] = i0 + 4
                    g0 = pltpu.async_copy(xh_hbm.at[i0b], r0, sg0)
                    g1 = pltpu.async_copy(xh_hbm.at[i1b], r1, sg1)
                    c0.wait()
                    c1.wait()
                    g0.wait()
                    g1.wait()

                    def edge(i, _):
                        i16 = (i >> 4) << 4
                        lane = jnp.full((16,), i & 15, jnp.int32)
                        a0 = jnp.take(alb[0, pl.ds(i16, 16)], lane)
                        a1 = jnp.take(alb[1, pl.ds(i16, 16)], lane)
                        for p2 in range(2):
                            sl = pl.ds(p2 * 16, 16)
                            msg[i, sl] = a0 * r0[i, sl] + a1 * r1[i, sl]
                        return 0
                    lax.fori_loop(0, 128, edge, 0)
                    cd.wait()
                    pltpu.sync_copy(msg, outs.at[dstb.at[0]], add=True)
                    return 0
                lax.fori_loop(0, pj, panel, 0)
            plsc.subcore_barrier()
            for z in range(16):
                rs = s * nps + z * zr
                pltpu.sync_copy(outs.at[pl.ds(rs, zr)], outq_hbm.at[q, pl.ds(rs, zr)])
            plsc.subcore_barrier()

    return k(xh_flat, src, dst, alpha)


def _gat_layer(xin, w, att, src, dst, ea, npad):
    xh, a = _dense(xin, w, att)
    alpha = _alpha_call(jnp.transpose(a, (2, 0, 1)), src, dst, ea)
    t = w.shape[0]
    xh_flat = xh.reshape(t * npad * 8, 32)
    return _message_call(xh_flat, src, dst, alpha, npad)


def kernel(x, edge_index_0, edge_index_1, edge_index_2, edge_index_3,
           edge_index_4, edge_index_5, edge_attr_0, edge_attr_1, edge_attr_2,
           edge_attr_3, edge_attr_4, edge_attr_5, W1, att1, W2, att2,
           gamma1, beta1, gamma2, beta2, query, W_out, b_out):
    n, d = x.shape
    eis = [edge_index_0, edge_index_1, edge_index_2, edge_index_3,
           edge_index_4, edge_index_5]
    eas = [edge_attr_0, edge_attr_1, edge_attr_2, edge_attr_3,
           edge_attr_4, edge_attr_5]
    e = eis[0].shape[1]
    npad = ((n + 255) // 256) * 256
    p = e // 128

    src = jnp.stack([ei[0] for ei in eis]).reshape(6, p, 128)
    dst = jnp.stack([ei[1] for ei in eis]).reshape(6, p, 128)
    ea = jnp.stack([a[:, 0] for a in eas]).reshape(6, p, 128)
    xp = jnp.pad(x, ((0, npad - n), (0, 0)))

    raw1 = _gat_layer(xp, W1, att1, src, dst, ea, npad)
    h = _norm(raw1, gamma1, beta1)
    raw2 = _gat_layer(h, W2, att2, src, dst, ea, npad)
    h2 = _norm(raw2, gamma2, beta2)
    return _pool(h2, query, W_out, b_out, n)
